# baseline (speedup 1.0000x reference)
import functools

import jax
import jax.numpy as jnp
from jax import lax
from jax.experimental import pallas as pl
from jax.experimental.pallas import tpu as pltpu
from jax.experimental.pallas import tpu_sc as plsc

N = 10000
E = 320000
D = 128
H = 8
C = 16
NC = 2
NS = 16
NW = NC * NS
EPT = E // NW
CHUNK = 128
NG = CHUNK // 16
NCHUNK = 79
EPTP = NCHUNK * CHUNK
NACC = 10008
NDEN = 640
SLAB = 624


def _lin_body(x_ref, wl_ref, bl_ref, wr_ref, br_ref, xl_ref, xr_ref):
    x = x_ref[...]
    xl_ref[...] = jnp.dot(x, wl_ref[...], preferred_element_type=jnp.float32) + bl_ref[...]
    xr_ref[...] = jnp.dot(x, wr_ref[...], preferred_element_type=jnp.float32) + br_ref[...]


def _linear(x, W_l, b_l, W_r, b_r):
    return pl.pallas_call(
        _lin_body,
        out_shape=(
            jax.ShapeDtypeStruct((N, D), jnp.float32),
            jax.ShapeDtypeStruct((N, D), jnp.float32),
        ),
    )(x, W_l, b_l.reshape(1, D), W_r, b_r.reshape(1, D))


def _edge_body(xl_hbm, xr_hbm, src_hbm, dst_hbm, dst16_hbm, ea_hbm,
               coef_hbm, out_hbm, outd_hbm,
               src_c, dst_c, dstg_c, dst16_c, ea_c, coef_v, xl_v, xr_v,
               acc, accd, sem1, sem2):
    cid = lax.axis_index("c")
    sid = lax.axis_index("s")
    wid = cid * NS + sid
    lane = lax.iota(jnp.int32, 16)
    perms = [lane ^ o for o in (8, 4, 2, 1)]
    zero16 = jnp.zeros((16,), jnp.float32)

    def lanesum(v):
        for p in perms:
            v = v + v.at[p].get(mode='promise_in_bounds')
        return v

    pltpu.sync_copy(coef_hbm, coef_v)

    def zrow(r, _):
        for j in range(D // 16):
            xl_v[r, pl.ds(16 * j, 16)] = zero16
        return 0
    lax.fori_loop(0, CHUNK, zrow, 0)
    row0 = sid * SLAB
    for t in range(4):
        pltpu.sync_copy(xl_v, acc.at[pl.ds(row0 + t * CHUNK, CHUNK)])
    pltpu.sync_copy(xl_v.at[pl.ds(0, 112)], acc.at[pl.ds(row0 + 512, 112)])
    pltpu.sync_copy(xl_v.at[pl.ds(0, 40)], accd.at[pl.ds(sid * 40, 40)])

    @pl.when(sid == NS - 1)
    def _():
        pltpu.sync_copy(xl_v.at[pl.ds(0, 24)], acc.at[pl.ds(NS * SLAB, 24)])

    plsc.subcore_barrier()

    wej = [coef_v[0, pl.ds(16 * j, 16)] for j in range(H)]
    attj = [coef_v[1, pl.ds(16 * j, 16)] for j in range(H)]
    rbs = [jnp.full((16,), u, jnp.int32) for u in range(16)]
    perm8 = lane ^ 8
    perm4 = lane ^ 4
    perm2 = lane ^ 2
    perm1 = lane ^ 1
    msel8 = lane < 8
    msel4 = ((lane >> 2) & 1) == 0
    msel2 = ((lane >> 1) & 1) == 0
    LHEAD = [0, 4, 2, 6, 8, 12, 10, 14]
    spl = [jnp.full((16,), LHEAD[j], jnp.int32) for j in range(H)]
    denidx = jnp.where(msel8,
                       (lane >> 2) * 8 + (lane & 1) * 4 + ((lane >> 1) & 1) * 2,
                       0)

    def g16(v, p):
        return v.at[p].get(mode='promise_in_bounds')

    def chunk_body(k, _):
        ebase = wid * EPTP + k * CHUNK
        pltpu.sync_copy(src_hbm.at[pl.ds(ebase, CHUNK)], src_c)
        pltpu.sync_copy(dst_hbm.at[pl.ds(ebase, CHUNK)], dst_c)
        pltpu.sync_copy(dst16_hbm.at[wid * NCHUNK + k], dst16_c)
        pltpu.sync_copy(ea_hbm.at[wid * NCHUNK + k], ea_c)
        cp1 = pltpu.async_copy(xl_hbm.at[src_c], xl_v, sem1)
        cp2 = pltpu.async_copy(xr_hbm.at[dst_c], xr_v, sem2)
        for g in range(NG):
            dstg_c[pl.ds(16 * g, 16)] = dst16_c[g] >> 4
        cp1.wait()
        cp2.wait()

        @plsc.parallel_loop(0, NG, 1)
        def group_body(g):
            eav = ea_c[g]
            dsv = dst16_c[g]
            for u in range(16):
                e = g * 16 + u
                ebc = eav.at[rbs[u]].get(mode='promise_in_bounds')
                dsp = dsv.at[rbs[u]].get(mode='promise_in_bounds')
                xls = []
                ps = []
                for j in range(H):
                    xlj = xl_v[e, pl.ds(16 * j, 16)]
                    xrj = xr_v[e, pl.ds(16 * j, 16)]
                    m = xlj + xrj + ebc * wej[j]
                    m = jnp.maximum(m, 0.2 * m)
                    xls.append(xlj)
                    ps.append(m * attj[j])
                a = [p + g16(p, perm8) for p in ps]
                r = [jnp.where(msel8, a[j], a[j + 4]) for j in range(4)]
                b4 = [v + g16(v, perm4) for v in r]
                t2 = [jnp.where(msel4, b4[0], b4[1]),
                      jnp.where(msel4, b4[2], b4[3])]
                u2 = [v + g16(v, perm2) for v in t2]
                w1 = jnp.where(msel2, u2[0], u2[1])
                q = w1 + g16(w1, perm1)
                exq = jnp.exp(q)
                for j in range(H):
                    xl_v[e, pl.ds(16 * j, 16)] = xls[j] * g16(exq, spl[j])
                denp = g16(exq, denidx)
                den = jnp.where(msel8, denp, zero16)
                den_hi = g16(den, perm8)
                oddf = (dsp & 1).astype(jnp.float32)
                drow = den + oddf * (den_hi - den)
                for t in range(D // 16):
                    xr_v[e, pl.ds(16 * t, 16)] = zero16
                sc = dsv[u]
                xr_v[e, pl.ds((sc & 14) * 8, 16)] = drow

        pltpu.sync_copy(xl_v, acc.at[dst_c], add=True)
        pltpu.sync_copy(xr_v, accd.at[dstg_c], add=True)
        return 0

    lax.fori_loop(0, NCHUNK, chunk_body, 0)
    plsc.subcore_barrier()

    for t in range(4):
        pltpu.sync_copy(acc.at[pl.ds(row0 + t * CHUNK, CHUNK)],
                        out_hbm.at[cid, pl.ds(row0 + t * CHUNK, CHUNK)])
    pltpu.sync_copy(acc.at[pl.ds(row0 + 512, 112)],
                    out_hbm.at[cid, pl.ds(row0 + 512, 112)])
    pltpu.sync_copy(accd.at[pl.ds(sid * 40, 40)],
                    outd_hbm.at[cid, pl.ds(sid * 40, 40)])

    @pl.when(sid == NS - 1)
    def _():
        pltpu.sync_copy(acc.at[pl.ds(NS * SLAB, 24)],
                    out_hbm.at[cid, pl.ds(NS * SLAB, 24)])


def _edge_stage(xl, xr, src, dst, dst16, ea, coef):
    mesh = plsc.VectorSubcoreMesh(core_axis_name="c", subcore_axis_name="s")
    fn = functools.partial(
        pl.kernel,
        mesh=mesh,
        out_type=(
            jax.ShapeDtypeStruct((NC, NACC, D), jnp.float32),
            jax.ShapeDtypeStruct((NC, NDEN, D), jnp.float32),
        ),
        scratch_types=[
            pltpu.VMEM((CHUNK,), jnp.int32),
            pltpu.VMEM((CHUNK,), jnp.int32),
            pltpu.VMEM((CHUNK,), jnp.int32),
            pltpu.VMEM((NG, 16), jnp.int32),
            pltpu.VMEM((NG, 16), jnp.float32),
            pltpu.VMEM((2, D), jnp.float32),
            pltpu.VMEM((CHUNK, D), jnp.float32),
            pltpu.VMEM((CHUNK, D), jnp.float32),
            pltpu.VMEM_SHARED((NACC, D), jnp.float32),
            pltpu.VMEM_SHARED((NDEN, D), jnp.float32),
            pltpu.SemaphoreType.DMA,
            pltpu.SemaphoreType.DMA,
        ],
    )(_edge_body)
    return fn(xl, xr, src, dst, dst16, ea, coef)


def _post_body(num_ref, den_ref, bias_ref, gamma_ref, beta_ref, x_ref, y_ref):
    num = num_ref[0] + num_ref[1]
    den8 = den_ref[0] + den_ref[1]
    head = lax.broadcasted_iota(jnp.int32, (H, D), 0)
    lanes = lax.broadcasted_iota(jnp.int32, (H, D), 1) // C
    sel = (head == lanes).astype(jnp.float32)
    den = jnp.dot(den8, sel, preferred_element_type=jnp.float32)
    out = num / (den + 1e-16) + bias_ref[...]
    mu = jnp.mean(out, axis=-1, keepdims=True)
    var = jnp.mean((out - mu) ** 2, axis=-1, keepdims=True)
    xn = (out - mu) * lax.rsqrt(var + 1e-5) * gamma_ref[...] + beta_ref[...]
    y_ref[...] = jnp.where(xn > 0, xn, jnp.exp(xn) - 1.0) + x_ref[...]


def _post(num, den, bias, gamma, beta, x):
    B = 1000
    return pl.pallas_call(
        _post_body,
        grid=(N // B,),
        in_specs=[
            pl.BlockSpec((NC, B, D), lambda i: (0, i, 0)),
            pl.BlockSpec((NC, B, H), lambda i: (0, i, 0)),
            pl.BlockSpec((1, D), lambda i: (0, 0)),
            pl.BlockSpec((1, D), lambda i: (0, 0)),
            pl.BlockSpec((1, D), lambda i: (0, 0)),
            pl.BlockSpec((B, D), lambda i: (i, 0)),
        ],
        out_specs=pl.BlockSpec((B, D), lambda i: (i, 0)),
        out_shape=jax.ShapeDtypeStruct((N, D), jnp.float32),
    )(num, den, bias, gamma, beta, x)


def kernel(x, edge_index, edge_attr, W_l, b_l, W_r, b_r, W_e, att, bias, gamma, beta):
    xl, xr = _linear(x, W_l, b_l, W_r, b_r)
    npad = EPTP - EPT
    src = jnp.concatenate(
        [edge_index[0].reshape(NW, EPT),
         jnp.zeros((NW, npad), jnp.int32)], axis=1).reshape(-1)
    dstp = jnp.concatenate(
        [edge_index[1].reshape(NW, EPT),
         jnp.full((NW, npad), N, jnp.int32)], axis=1)
    dst = dstp.reshape(-1)
    dst16 = dstp.reshape(NW * NCHUNK, NG, 16)
    ea = jnp.concatenate(
        [edge_attr.reshape(NW, EPT),
         jnp.zeros((NW, npad), jnp.float32)], axis=1).reshape(NW * NCHUNK, NG, 16)
    coef = jnp.stack([W_e.reshape(D), att.reshape(D)])
    accs, dena = _edge_stage(xl, xr, src, dst, dst16, ea, coef)
    num = accs[:, :N, :]
    den = dena[:, :N // 16, :].reshape(NC, N, H)
    return _post(num, den, bias.reshape(1, D), gamma.reshape(1, D),
                 beta.reshape(1, D), x)


# speedup vs baseline: 1.0030x; 1.0030x over previous
import functools

import jax
import jax.numpy as jnp
from jax import lax
from jax.experimental import pallas as pl
from jax.experimental.pallas import tpu as pltpu
from jax.experimental.pallas import tpu_sc as plsc

N = 10000
E = 320000
D = 128
H = 8
C = 16
NC = 2
NS = 16
NW = NC * NS
EPT = E // NW
CHUNK = 80
NG = CHUNK // 16
NCHUNK = EPT // CHUNK
NDEN = 640
SLAB = 624


def _lin_body(x_ref, wl_ref, bl_ref, wr_ref, br_ref, xl_ref, xr_ref):
    x = x_ref[...]
    xl_ref[...] = jnp.dot(x, wl_ref[...], preferred_element_type=jnp.float32) + bl_ref[...]
    xr_ref[...] = jnp.dot(x, wr_ref[...], preferred_element_type=jnp.float32) + br_ref[...]


def _linear(x, W_l, b_l, W_r, b_r):
    return pl.pallas_call(
        _lin_body,
        out_shape=(
            jax.ShapeDtypeStruct((N, D), jnp.float32),
            jax.ShapeDtypeStruct((N, D), jnp.float32),
        ),
    )(x, W_l, b_l.reshape(1, D), W_r, b_r.reshape(1, D))


def _edge_body(xl_hbm, xr_hbm, src_hbm, dst_hbm, dst16_hbm, ea_hbm,
               coef_hbm, out_hbm, outd_hbm,
               src_c, dst_c, dstg_c, dst16_c, ea_c, coef_v, xl_v, xr_v,
               od_v, acc, accd, sem1, sem2):
    cid = lax.axis_index("c")
    sid = lax.axis_index("s")
    wid = cid * NS + sid
    lane = lax.iota(jnp.int32, 16)
    perms = [lane ^ o for o in (8, 4, 2, 1)]
    zero16 = jnp.zeros((16,), jnp.float32)

    def lanesum(v):
        for p in perms:
            v = v + v.at[p].get(mode='promise_in_bounds')
        return v

    pltpu.sync_copy(coef_hbm, coef_v)

    def zrow(r, _):
        for j in range(D // 16):
            xl_v[r, pl.ds(16 * j, 16)] = zero16
            od_v[r, pl.ds(16 * j, 16)] = zero16
        return 0
    lax.fori_loop(0, CHUNK, zrow, 0)
    row0 = sid * SLAB
    for t in range(7):
        pltpu.sync_copy(xl_v, acc.at[pl.ds(row0 + t * CHUNK, CHUNK)])
    pltpu.sync_copy(xl_v.at[pl.ds(0, 64)], acc.at[pl.ds(row0 + 560, 64)])
    pltpu.sync_copy(xl_v.at[pl.ds(0, 40)], accd.at[pl.ds(sid * 40, 40)])

    @pl.when(sid == NS - 1)
    def _():
        pltpu.sync_copy(xl_v.at[pl.ds(0, 16)], acc.at[pl.ds(NS * SLAB, 16)])

    plsc.subcore_barrier()

    wej = [coef_v[0, pl.ds(16 * j, 16)] for j in range(H)]
    attj = [coef_v[1, pl.ds(16 * j, 16)] for j in range(H)]
    rbs = [jnp.full((16,), u, jnp.int32) for u in range(16)]
    perm8 = lane ^ 8
    perm4 = lane ^ 4
    perm2 = lane ^ 2
    perm1 = lane ^ 1
    msel8 = lane < 8
    msel4 = ((lane >> 2) & 1) == 0
    msel2 = ((lane >> 1) & 1) == 0
    LHEAD = [0, 4, 2, 6, 8, 12, 10, 14]
    spl = [jnp.full((16,), LHEAD[j], jnp.int32) for j in range(H)]
    denidx = jnp.where(msel8,
                       (lane >> 2) * 8 + (lane & 1) * 4 + ((lane >> 1) & 1) * 2,
                       0)

    def g16(v, p):
        return v.at[p].get(mode='promise_in_bounds')

    def chunk_body(k, _):
        ebase = wid * EPT + k * CHUNK
        pltpu.sync_copy(src_hbm.at[pl.ds(ebase, CHUNK)], src_c)
        pltpu.sync_copy(dst_hbm.at[pl.ds(ebase, CHUNK)], dst_c)
        pltpu.sync_copy(dst16_hbm.at[wid * NCHUNK + k], dst16_c)
        pltpu.sync_copy(ea_hbm.at[wid * NCHUNK + k], ea_c)
        cp1 = pltpu.async_copy(xl_hbm.at[src_c], xl_v, sem1)
        cp2 = pltpu.async_copy(xr_hbm.at[dst_c], xr_v, sem2)
        for g in range(NG):
            dstg_c[pl.ds(16 * g, 16)] = dst16_c[g] >> 4
        cp1.wait()
        cp2.wait()

        @plsc.parallel_loop(0, NG, 1)
        def group_body(g):
            eav = ea_c[g]
            dsv = dst16_c[g]
            for u in range(16):
                e = g * 16 + u
                ebc = eav.at[rbs[u]].get(mode='promise_in_bounds')
                dsp = dsv.at[rbs[u]].get(mode='promise_in_bounds')
                xls = []
                ps = []
                for j in range(H):
                    xlj = xl_v[e, pl.ds(16 * j, 16)]
                    xrj = xr_v[e, pl.ds(16 * j, 16)]
                    m = xlj + xrj + ebc * wej[j]
                    m = jnp.maximum(m, 0.2 * m)
                    xls.append(xlj)
                    ps.append(m * attj[j])
                a = [p + g16(p, perm8) for p in ps]
                r = [jnp.where(msel8, a[j], a[j + 4]) for j in range(4)]
                b4 = [v + g16(v, perm4) for v in r]
                t2 = [jnp.where(msel4, b4[0], b4[1]),
                      jnp.where(msel4, b4[2], b4[3])]
                u2 = [v + g16(v, perm2) for v in t2]
                w1 = jnp.where(msel2, u2[0], u2[1])
                q = w1 + g16(w1, perm1)
                exq = jnp.exp(q)
                for j in range(H):
                    xl_v[e, pl.ds(16 * j, 16)] = xls[j] * g16(exq, spl[j])
                denp = g16(exq, denidx)
                den = jnp.where(msel8, denp, zero16)
                drow = g16(den, lane ^ ((dsp & 1) * 8))
                sc = dsv[u]
                od_v[e, pl.ds((sc & 14) * 8, 16)] = drow

        pltpu.sync_copy(xl_v, acc.at[dst_c], add=True)
        pltpu.sync_copy(od_v, accd.at[dstg_c], add=True)

        @plsc.parallel_loop(0, NG, 1)
        def rezero_body(g):
            dsv = dst16_c[g]
            for u in range(16):
                sc = dsv[u]
                od_v[g * 16 + u, pl.ds((sc & 14) * 8, 16)] = zero16
        return 0

    lax.fori_loop(0, NCHUNK, chunk_body, 0)
    plsc.subcore_barrier()

    for t in range(7):
        pltpu.sync_copy(acc.at[pl.ds(row0 + t * CHUNK, CHUNK)],
                        out_hbm.at[cid, pl.ds(row0 + t * CHUNK, CHUNK)])
    pltpu.sync_copy(acc.at[pl.ds(row0 + 560, 64)],
                    out_hbm.at[cid, pl.ds(row0 + 560, 64)])
    pltpu.sync_copy(accd.at[pl.ds(sid * 40, 40)],
                    outd_hbm.at[cid, pl.ds(sid * 40, 40)])

    @pl.when(sid == NS - 1)
    def _():
        pltpu.sync_copy(acc.at[pl.ds(NS * SLAB, 16)],
                        out_hbm.at[cid, pl.ds(NS * SLAB, 16)])


def _edge_stage(xl, xr, src, dst, dst16, ea, coef):
    mesh = plsc.VectorSubcoreMesh(core_axis_name="c", subcore_axis_name="s")
    fn = functools.partial(
        pl.kernel,
        mesh=mesh,
        out_type=(
            jax.ShapeDtypeStruct((NC, N, D), jnp.float32),
            jax.ShapeDtypeStruct((NC, NDEN, D), jnp.float32),
        ),
        scratch_types=[
            pltpu.VMEM((CHUNK,), jnp.int32),
            pltpu.VMEM((CHUNK,), jnp.int32),
            pltpu.VMEM((CHUNK,), jnp.int32),
            pltpu.VMEM((NG, 16), jnp.int32),
            pltpu.VMEM((NG, 16), jnp.float32),
            pltpu.VMEM((2, D), jnp.float32),
            pltpu.VMEM((CHUNK, D), jnp.float32),
            pltpu.VMEM((CHUNK, D), jnp.float32),
            pltpu.VMEM((CHUNK, D), jnp.float32),
            pltpu.VMEM_SHARED((N, D), jnp.float32),
            pltpu.VMEM_SHARED((NDEN, D), jnp.float32),
            pltpu.SemaphoreType.DMA,
            pltpu.SemaphoreType.DMA,
        ],
    )(_edge_body)
    return fn(xl, xr, src, dst, dst16, ea, coef)


def _post_body(num_ref, den_ref, bias_ref, gamma_ref, beta_ref, x_ref, y_ref):
    num = num_ref[0] + num_ref[1]
    den8 = den_ref[0] + den_ref[1]
    head = lax.broadcasted_iota(jnp.int32, (H, D), 0)
    lanes = lax.broadcasted_iota(jnp.int32, (H, D), 1) // C
    sel = (head == lanes).astype(jnp.float32)
    den = jnp.dot(den8, sel, preferred_element_type=jnp.float32)
    out = num / (den + 1e-16) + bias_ref[...]
    mu = jnp.mean(out, axis=-1, keepdims=True)
    var = jnp.mean((out - mu) ** 2, axis=-1, keepdims=True)
    xn = (out - mu) * lax.rsqrt(var + 1e-5) * gamma_ref[...] + beta_ref[...]
    y_ref[...] = jnp.where(xn > 0, xn, jnp.exp(xn) - 1.0) + x_ref[...]


def _post(num, den, bias, gamma, beta, x):
    B = 1000
    return pl.pallas_call(
        _post_body,
        grid=(N // B,),
        in_specs=[
            pl.BlockSpec((NC, B, D), lambda i: (0, i, 0)),
            pl.BlockSpec((NC, B, H), lambda i: (0, i, 0)),
            pl.BlockSpec((1, D), lambda i: (0, 0)),
            pl.BlockSpec((1, D), lambda i: (0, 0)),
            pl.BlockSpec((1, D), lambda i: (0, 0)),
            pl.BlockSpec((B, D), lambda i: (i, 0)),
        ],
        out_specs=pl.BlockSpec((B, D), lambda i: (i, 0)),
        out_shape=jax.ShapeDtypeStruct((N, D), jnp.float32),
    )(num, den, bias, gamma, beta, x)


def kernel(x, edge_index, edge_attr, W_l, b_l, W_r, b_r, W_e, att, bias, gamma, beta):
    xl, xr = _linear(x, W_l, b_l, W_r, b_r)
    src = edge_index[0]
    dst = edge_index[1]
    dst16 = dst.reshape(E // CHUNK, NG, 16)
    ea = edge_attr.reshape(E // CHUNK, NG, 16)
    coef = jnp.stack([W_e.reshape(D), att.reshape(D)])
    num, dena = _edge_stage(xl, xr, src, dst, dst16, ea, coef)
    den = dena[:, :N // 16, :].reshape(NC, N, H)
    return _post(num, den, bias.reshape(1, D), gamma.reshape(1, D),
                 beta.reshape(1, D), x)


# speedup vs baseline: 1.0126x; 1.0096x over previous
import functools

import jax
import jax.numpy as jnp
from jax import lax
from jax.experimental import pallas as pl
from jax.experimental.pallas import tpu as pltpu
from jax.experimental.pallas import tpu_sc as plsc

N = 10000
E = 320000
D = 128
H = 8
C = 16
NC = 2
NS = 16
NW = NC * NS
EPT = E // NW
CHUNK = 80
NG = CHUNK // 16
NCHUNK = EPT // CHUNK
NDEN = 640
SLAB = 624


def _lin_body(x_ref, wl_ref, bl_ref, wr_ref, br_ref, xl_ref, xr_ref):
    x = x_ref[...]
    xl_ref[...] = jnp.dot(x, wl_ref[...], preferred_element_type=jnp.float32) + bl_ref[...]
    xr_ref[...] = jnp.dot(x, wr_ref[...], preferred_element_type=jnp.float32) + br_ref[...]


def _linear(x, W_l, b_l, W_r, b_r):
    return pl.pallas_call(
        _lin_body,
        out_shape=(
            jax.ShapeDtypeStruct((N, D), jnp.float32),
            jax.ShapeDtypeStruct((N, D), jnp.float32),
        ),
    )(x, W_l, b_l.reshape(1, D), W_r, b_r.reshape(1, D))


def _edge_body(xl_hbm, xr_hbm, src_hbm, dst_hbm, dst16_hbm, ea_hbm,
               coef_hbm, out_hbm, outd_hbm,
               src_c, dst_c, dstg_c, dst16_c, ea_c, coef_v, xl_v, xr_v,
               acc, accd, sem1, sem2):
    cid = lax.axis_index("c")
    sid = lax.axis_index("s")
    wid = cid * NS + sid
    lane = lax.iota(jnp.int32, 16)
    zero16 = jnp.zeros((16,), jnp.float32)

    pltpu.sync_copy(coef_hbm, coef_v)

    def zrow(r, _):
        for j in range(D // 16):
            xl_v[r, pl.ds(16 * j, 16)] = zero16
        return 0
    lax.fori_loop(0, CHUNK, zrow, 0)
    row0 = sid * SLAB
    for t in range(7):
        pltpu.sync_copy(xl_v, acc.at[pl.ds(row0 + t * CHUNK, CHUNK)])
    pltpu.sync_copy(xl_v.at[pl.ds(0, 64)], acc.at[pl.ds(row0 + 560, 64)])
    pltpu.sync_copy(xl_v.at[pl.ds(0, 40)], accd.at[pl.ds(sid * 40, 40)])

    @pl.when(sid == NS - 1)
    def _():
        pltpu.sync_copy(xl_v.at[pl.ds(0, 16)], acc.at[pl.ds(NS * SLAB, 16)])

    plsc.subcore_barrier()

    wej = [coef_v[0, pl.ds(16 * j, 16)] for j in range(H)]
    attj = [coef_v[1, pl.ds(16 * j, 16)] for j in range(H)]
    rbs = [jnp.full((16,), u, jnp.int32) for u in range(16)]
    perm8 = lane ^ 8
    perm4 = lane ^ 4
    perm2 = lane ^ 2
    perm1 = lane ^ 1
    msel8 = lane < 8
    msel4 = ((lane >> 2) & 1) == 0
    msel2 = ((lane >> 1) & 1) == 0
    LHEAD = [0, 4, 2, 6, 8, 12, 10, 14]
    spl = [jnp.full((16,), LHEAD[j], jnp.int32) for j in range(H)]
    denidx = jnp.where(msel8,
                       (lane >> 2) * 8 + (lane & 1) * 4 + ((lane >> 1) & 1) * 2,
                       0)

    def g16(v, p):
        return v.at[p].get(mode='promise_in_bounds')

    def chunk_body(k, _):
        ebase = wid * EPT + k * CHUNK
        pltpu.sync_copy(src_hbm.at[pl.ds(ebase, CHUNK)], src_c)
        pltpu.sync_copy(dst_hbm.at[pl.ds(ebase, CHUNK)], dst_c)
        pltpu.sync_copy(dst16_hbm.at[wid * NCHUNK + k], dst16_c)
        pltpu.sync_copy(ea_hbm.at[wid * NCHUNK + k], ea_c)
        cp1 = pltpu.async_copy(xl_hbm.at[src_c], xl_v, sem1)
        cp2 = pltpu.async_copy(xr_hbm.at[dst_c], xr_v, sem2)
        for g in range(NG):
            dstg_c[pl.ds(16 * g, 16)] = dst16_c[g] >> 4
        cp1.wait()
        cp2.wait()

        @plsc.parallel_loop(0, NG, 1)
        def group_body(g):
            eav = ea_c[g]
            dsv = dst16_c[g]
            for u in range(16):
                e = g * 16 + u
                ebc = eav.at[rbs[u]].get(mode='promise_in_bounds')
                dsp = dsv.at[rbs[u]].get(mode='promise_in_bounds')
                xls = []
                ps = []
                for j in range(H):
                    xlj = xl_v[e, pl.ds(16 * j, 16)]
                    xrj = xr_v[e, pl.ds(16 * j, 16)]
                    m = xlj + xrj + ebc * wej[j]
                    m = jnp.maximum(m, 0.2 * m)
                    xls.append(xlj)
                    ps.append(m * attj[j])
                a = [p + g16(p, perm8) for p in ps]
                r = [jnp.where(msel8, a[j], a[j + 4]) for j in range(4)]
                b4 = [v + g16(v, perm4) for v in r]
                t2 = [jnp.where(msel4, b4[0], b4[1]),
                      jnp.where(msel4, b4[2], b4[3])]
                u2 = [v + g16(v, perm2) for v in t2]
                w1 = jnp.where(msel2, u2[0], u2[1])
                q = w1 + g16(w1, perm1)
                exq = jnp.exp(q)
                for j in range(H):
                    xl_v[e, pl.ds(16 * j, 16)] = xls[j] * g16(exq, spl[j])
                denp = g16(exq, denidx)
                den = jnp.where(msel8, denp, zero16)
                den_hi = g16(den, perm8)
                oddf = (dsp & 1).astype(jnp.float32)
                drow = den + oddf * (den_hi - den)
                for t in range(D // 16):
                    xr_v[e, pl.ds(16 * t, 16)] = zero16
                sc = dsv[u]
                xr_v[e, pl.ds((sc & 14) * 8, 16)] = drow

        pltpu.sync_copy(xl_v, acc.at[dst_c], add=True)
        pltpu.sync_copy(xr_v, accd.at[dstg_c], add=True)
        return 0

    lax.fori_loop(0, NCHUNK, chunk_body, 0)
    plsc.subcore_barrier()

    for t in range(7):
        pltpu.sync_copy(acc.at[pl.ds(row0 + t * CHUNK, CHUNK)],
                        out_hbm.at[cid, pl.ds(row0 + t * CHUNK, CHUNK)])
    pltpu.sync_copy(acc.at[pl.ds(row0 + 560, 64)],
                    out_hbm.at[cid, pl.ds(row0 + 560, 64)])
    pltpu.sync_copy(accd.at[pl.ds(sid * 40, 40)],
                    outd_hbm.at[cid, pl.ds(sid * 40, 40)])

    @pl.when(sid == NS - 1)
    def _():
        pltpu.sync_copy(acc.at[pl.ds(NS * SLAB, 16)],
                        out_hbm.at[cid, pl.ds(NS * SLAB, 16)])


def _edge_stage(xl, xr, src, dst, dst16, ea, coef):
    mesh = plsc.VectorSubcoreMesh(core_axis_name="c", subcore_axis_name="s")
    fn = functools.partial(
        pl.kernel,
        mesh=mesh,
        out_type=(
            jax.ShapeDtypeStruct((NC, N, D), jnp.float32),
            jax.ShapeDtypeStruct((NC, NDEN, D), jnp.float32),
        ),
        scratch_types=[
            pltpu.VMEM((CHUNK,), jnp.int32),
            pltpu.VMEM((CHUNK,), jnp.int32),
            pltpu.VMEM((CHUNK,), jnp.int32),
            pltpu.VMEM((NG, 16), jnp.int32),
            pltpu.VMEM((NG, 16), jnp.float32),
            pltpu.VMEM((2, D), jnp.float32),
            pltpu.VMEM((CHUNK, D), jnp.float32),
            pltpu.VMEM((CHUNK, D), jnp.float32),
            pltpu.VMEM_SHARED((N, D), jnp.float32),
            pltpu.VMEM_SHARED((NDEN, D), jnp.float32),
            pltpu.SemaphoreType.DMA,
            pltpu.SemaphoreType.DMA,
        ],
    )(_edge_body)
    return fn(xl, xr, src, dst, dst16, ea, coef)


def _post_body(num_ref, den_ref, bias_ref, gamma_ref, beta_ref, x_ref, y_ref):
    num = num_ref[0] + num_ref[1]
    den8 = den_ref[0] + den_ref[1]
    head = lax.broadcasted_iota(jnp.int32, (H, D), 0)
    lanes = lax.broadcasted_iota(jnp.int32, (H, D), 1) // C
    sel = (head == lanes).astype(jnp.float32)
    den = jnp.dot(den8, sel, preferred_element_type=jnp.float32)
    out = num / (den + 1e-16) + bias_ref[...]
    mu = jnp.mean(out, axis=-1, keepdims=True)
    var = jnp.mean((out - mu) ** 2, axis=-1, keepdims=True)
    xn = (out - mu) * lax.rsqrt(var + 1e-5) * gamma_ref[...] + beta_ref[...]
    y_ref[...] = jnp.where(xn > 0, xn, jnp.exp(xn) - 1.0) + x_ref[...]


def _post(num, den, bias, gamma, beta, x):
    B = 1000
    return pl.pallas_call(
        _post_body,
        grid=(N // B,),
        in_specs=[
            pl.BlockSpec((NC, B, D), lambda i: (0, i, 0)),
            pl.BlockSpec((NC, B, H), lambda i: (0, i, 0)),
            pl.BlockSpec((1, D), lambda i: (0, 0)),
            pl.BlockSpec((1, D), lambda i: (0, 0)),
            pl.BlockSpec((1, D), lambda i: (0, 0)),
            pl.BlockSpec((B, D), lambda i: (i, 0)),
        ],
        out_specs=pl.BlockSpec((B, D), lambda i: (i, 0)),
        out_shape=jax.ShapeDtypeStruct((N, D), jnp.float32),
    )(num, den, bias, gamma, beta, x)


def kernel(x, edge_index, edge_attr, W_l, b_l, W_r, b_r, W_e, att, bias, gamma, beta):
    xl, xr = _linear(x, W_l, b_l, W_r, b_r)
    src = edge_index[0]
    dst = edge_index[1]
    dst16 = dst.reshape(E // CHUNK, NG, 16)
    ea = edge_attr.reshape(E // CHUNK, NG, 16)
    coef = jnp.stack([W_e.reshape(D), att.reshape(D)])
    num, dena = _edge_stage(xl, xr, src, dst, dst16, ea, coef)
    den = dena[:, :N // 16, :].reshape(NC, N, H)
    return _post(num, den, bias.reshape(1, D), gamma.reshape(1, D),
                 beta.reshape(1, D), x)
